# Initial kernel scaffold; baseline (speedup 1.0000x reference)
import jax
import jax.numpy as jnp
from jax.experimental import pallas as pl


def kernel(x, edge_index, params):
    raise NotImplementedError("write your pallas kernel here")


# speedup vs baseline: 26.3581x; 26.3581x over previous
import functools

import jax
import jax.numpy as jnp
from jax import lax
from jax.experimental import pallas as pl
from jax.experimental.pallas import tpu as pltpu
from jax.experimental.pallas import tpu_sc as plsc

N_NODES = 10000
N_PAD = 10240
HID = 32
NF = N_PAD // 4
N_WORKERS = 32
MC = 128
CHUNKS = 88
E_PAD = N_WORKERS * MC * CHUNKS
N_TILES = 16
ROWS_PER_TILE = N_PAD // N_TILES

_f32 = jnp.float32


def _attn_tabs(h_fl, amat, out_s, out_d, out_b):
    sc = jnp.dot(h_fl, amat, preferred_element_type=_f32)
    a_s = sc[:, 0:4]
    a_d = sc[:, 4:8]
    z = jnp.max(a_s) + a_d
    out_s[...] = a_s
    out_d[...] = a_d
    out_b[...] = jnp.maximum(z, 0.2 * z)


@functools.partial(
    pl.pallas_call,
    out_shape=(
        jax.ShapeDtypeStruct((NF, 128), _f32),
        jax.ShapeDtypeStruct((NF, 128), _f32),
    ) + tuple(jax.ShapeDtypeStruct((NF, 4), _f32) for _ in range(6)),
)
def _tc_pre(x_ref, wc_ref, wr_ref, ac_ref, ar_ref, hc_ref, hr_ref,
            asc_ref, adc_ref, bc_ref, asr_ref, adr_ref, br_ref):
    x = x_ref[...]
    hc = jnp.dot(x, wc_ref[...], preferred_element_type=_f32)
    hr = jnp.dot(x, wr_ref[...], preferred_element_type=_f32)
    hc_ref[...] = hc
    hr_ref[...] = hr
    _attn_tabs(hc, ac_ref[...], asc_ref, adc_ref, bc_ref)
    _attn_tabs(hr, ar_ref[...], asr_ref, adr_ref, br_ref)


@functools.partial(
    pl.pallas_call,
    out_shape=(
        jax.ShapeDtypeStruct((NF, 128), _f32),
        jax.ShapeDtypeStruct((NF, 128), _f32),
    ) + tuple(jax.ShapeDtypeStruct((NF, 4), _f32) for _ in range(6)),
)
def _tc_mid(pc_ref, dc_ref, pr_ref, dr_ref, b1_ref, w2c_ref, w2r_ref,
            ac_ref, ar_ref,
            h2c_ref, h2r_ref, asc_ref, adc_ref, bc_ref, asr_ref, adr_ref, br_ref):
    b1 = b1_ref[...]
    for i, p_ref, d_ref, w_ref, a_ref, h_o, ts, td, tb in (
            (0, pc_ref, dc_ref, w2c_ref, ac_ref, h2c_ref, asc_ref, adc_ref, bc_ref),
            (1, pr_ref, dr_ref, w2r_ref, ar_ref, h2r_ref, asr_ref, adr_ref, br_ref)):
        p = p_ref[...]
        d = d_ref[...]
        o = (p[0] + p[1]) / (d[0] + d[1] + 1e-16)
        g = jnp.maximum(o + b1[i:i + 1], 0.0)
        h2 = jnp.dot(g, w_ref[...], preferred_element_type=_f32)
        h_o[...] = h2
        _attn_tabs(h2, a_ref[...], ts, td, tb)


@functools.partial(
    pl.pallas_call,
    out_shape=(
        jax.ShapeDtypeStruct((NF, 128), _f32),
        jax.ShapeDtypeStruct((NF, 128), _f32),
    ),
)
def _tc_post(pc_ref, dc_ref, pr_ref, dr_ref, b2_ref, lwc_ref, lwr_ref, lb_ref,
             yc_ref, yr_ref):
    b2 = b2_ref[...]
    lb = lb_ref[...]
    for i, p_ref, d_ref, w_ref, y_ref in ((0, pc_ref, dc_ref, lwc_ref, yc_ref),
                                          (1, pr_ref, dr_ref, lwr_ref, yr_ref)):
        p = p_ref[...]
        d = d_ref[...]
        o = (p[0] + p[1]) / (d[0] + d[1] + 1e-16)
        g = jnp.maximum(o + b2[i:i + 1], 0.0)
        y = jnp.dot(g, w_ref[...], preferred_element_type=_f32) + lb[i:i + 1]
        if i == 0:
            y = jax.nn.sigmoid(y)
        y_ref[...] = y


_SC_OUT_TYPE = (
    jax.ShapeDtypeStruct((2, N_PAD, HID), _f32),
    jax.ShapeDtypeStruct((2, N_PAD, HID), _f32),
    jax.ShapeDtypeStruct((2, N_PAD, HID), _f32),
    jax.ShapeDtypeStruct((2, N_PAD, HID), _f32),
)

_SC_SCRATCH = [
    pltpu.VMEM((CHUNKS, MC), jnp.int32),
    pltpu.VMEM((CHUNKS, MC), jnp.int32),
    pltpu.VMEM((MC,), _f32),
    pltpu.VMEM((MC,), _f32),
    pltpu.VMEM((MC,), _f32),
    pltpu.VMEM((MC,), _f32),
    pltpu.VMEM((MC,), _f32),
    pltpu.VMEM((MC,), _f32),
    pltpu.VMEM((MC, HID), _f32),
    pltpu.VMEM((MC, HID), _f32),
    pltpu.VMEM((MC, HID), _f32),
    pltpu.VMEM((MC, HID), _f32),
    pltpu.VMEM_SHARED((N_PAD, HID), _f32),
    pltpu.VMEM_SHARED((N_PAD, HID), _f32),
    pltpu.VMEM_SHARED((N_PAD, HID), _f32),
    pltpu.VMEM_SHARED((N_PAD, HID), _f32),
    pltpu.SemaphoreType.DMA,
]

_G_DNUMS = lax.GatherDimensionNumbers(
    offset_dims=(), collapsed_slice_dims=(0,), start_index_map=(0,))


def _sc_edge_body(srcr, dstr, hc, asc, adc, bc, hr, asr, adr, br, z2,
                  pc, dc, pr, dr,
                  srcv, dstv, asec, adec, bec, aser, ader, ber,
                  wdc, wdr, hrowc, hrowr, accc, denc, accr, denr, sem):
    cid = lax.axis_index("c")
    sid = lax.axis_index("s")
    wid = sid * 2 + cid
    r0 = sid * ROWS_PER_TILE

    for acc in (accc, denc, accr, denr):
        pltpu.sync_copy(z2.at[pl.ds(r0, ROWS_PER_TILE)], acc.at[pl.ds(r0, ROWS_PER_TILE)])
    plsc.subcore_barrier()

    pltpu.sync_copy(srcr.at[pl.ds(wid * CHUNKS, CHUNKS)], srcv)
    pltpu.sync_copy(dstr.at[pl.ds(wid * CHUNKS, CHUNKS)], dstv)
    pltpu.sync_copy(z2.at[pl.ds(0, MC)], wdc)
    pltpu.sync_copy(z2.at[pl.ds(0, MC)], wdr)

    def _chunk(j, carry):
        si = srcv.at[j]
        di = dstv.at[j]
        cps = [
            pltpu.async_copy(asc.at[si], asec, sem),
            pltpu.async_copy(adc.at[di], adec, sem),
            pltpu.async_copy(bc.at[di], bec, sem),
            pltpu.async_copy(hc.at[si], hrowc, sem),
            pltpu.async_copy(asr.at[si], aser, sem),
            pltpu.async_copy(adr.at[di], ader, sem),
            pltpu.async_copy(br.at[di], ber, sem),
            pltpu.async_copy(hr.at[si], hrowr, sem),
        ]
        for cp in cps:
            cp.wait()
        for ase, ade, be, wd, hrow, acc, den in (
                (asec, adec, bec, wdc, hrowc, accc, denc),
                (aser, ader, ber, wdr, hrowr, accr, denr)):

            def _grp(kk, c2, ase=ase, ade=ade, be=be, wd=wd, hrow=hrow):
                s_ = pl.ds(kk * 16, 16)
                z = ase[s_] + ade[s_]
                z = jnp.maximum(z, 0.2 * z) - be[s_]
                w = jnp.exp(z)
                for i in range(16):
                    r = kk * 16 + i
                    wb = lax.gather(w, jnp.full((16, 1), i, jnp.int32), _G_DNUMS,
                                    (1,), mode=lax.GatherScatterMode.PROMISE_IN_BOUNDS)
                    wd[r, pl.ds(0, 16)] = wb + wd[r, pl.ds(0, 16)] * 0.0
                    wd[r, pl.ds(16, 16)] = wb + wd[r, pl.ds(16, 16)] * 0.0
                    hrow[r, pl.ds(0, 16)] = hrow[r, pl.ds(0, 16)] * wb
                    hrow[r, pl.ds(16, 16)] = hrow[r, pl.ds(16, 16)] * wb
                return c2

            lax.fori_loop(0, MC // 16, _grp, 0)
            pltpu.sync_copy(hrow, acc.at[di], add=True)
            pltpu.sync_copy(wd, den.at[di], add=True)
        return carry

    lax.fori_loop(0, CHUNKS, _chunk, 0)
    plsc.subcore_barrier()

    for acc, out in ((accc, pc), (denc, dc), (accr, pr), (denr, dr)):
        pltpu.sync_copy(acc.at[pl.ds(r0, ROWS_PER_TILE)],
                        out.at[cid, pl.ds(r0, ROWS_PER_TILE)])


@functools.cache
def _sc_edge():
    mesh = plsc.VectorSubcoreMesh(core_axis_name="c", subcore_axis_name="s",
                                  num_cores=2, num_subcores=N_TILES)
    return pl.kernel(_sc_edge_body, out_type=_SC_OUT_TYPE, mesh=mesh,
                     scratch_types=_SC_SCRATCH,
                     compiler_params=pltpu.CompilerParams(use_tc_tiling_on_sc=False))


def _bd(w):
    return jnp.kron(jnp.eye(4, dtype=_f32), w)


def _amat(att_src, att_dst):
    e4 = jnp.eye(4, dtype=_f32)
    return jnp.concatenate([jnp.kron(e4, att_src[:, None]),
                            jnp.kron(e4, att_dst[:, None])], axis=1)


def kernel(x, edge_index, params):
    pcl, prg = params['cls'], params['reg']

    xp = jnp.pad(x, ((0, N_PAD - N_NODES), (0, 0))).reshape(NF, 512)
    loops = jnp.arange(N_NODES, dtype=jnp.int32)
    n_fill = E_PAD - N_NODES - edge_index.shape[1]
    fill = jnp.full((n_fill,), N_NODES, jnp.int32)
    srcr = jnp.concatenate([edge_index[0], loops, fill]).reshape(E_PAD // MC, MC)
    dstr = jnp.concatenate([edge_index[1], loops, fill]).reshape(E_PAD // MC, MC)
    z2 = jnp.zeros((N_PAD, HID), _f32)
    sq = lambda t: t.reshape(N_PAD)
    fl = lambda t: t.reshape(2, NF, 128)

    w1c = jnp.kron(jnp.eye(4, dtype=_f32), pcl['W1'])
    w1r = jnp.kron(jnp.eye(4, dtype=_f32), prg['W1'])
    a1c = _amat(pcl['att_src1'], pcl['att_dst1'])
    a1r = _amat(prg['att_src1'], prg['att_dst1'])
    hc, hr, asc, adc, bc, asr, adr, br = _tc_pre(xp, w1c, w1r, a1c, a1r)

    pc, dc, pr, dr = _sc_edge()(srcr, dstr,
                                hc.reshape(N_PAD, HID), sq(asc), sq(adc), sq(bc),
                                hr.reshape(N_PAD, HID), sq(asr), sq(adr), sq(br), z2)

    b1 = jnp.stack([jnp.tile(pcl['b1'], 4), jnp.tile(prg['b1'], 4)])
    h2c, h2r, asc, adc, bc, asr, adr, br = _tc_mid(
        fl(pc), fl(dc), fl(pr), fl(dr), b1, _bd(pcl['W2']), _bd(prg['W2']),
        _amat(pcl['att_src2'], pcl['att_dst2']),
        _amat(prg['att_src2'], prg['att_dst2']))

    pc, dc, pr, dr = _sc_edge()(srcr, dstr,
                                h2c.reshape(N_PAD, HID), sq(asc), sq(adc), sq(bc),
                                h2r.reshape(N_PAD, HID), sq(asr), sq(adr), sq(br), z2)

    b2 = jnp.stack([jnp.tile(pcl['b2'], 4), jnp.tile(prg['b2'], 4)])
    lb = jnp.stack([jnp.tile(pcl['lin_b'], 4), jnp.tile(prg['lin_b'], 4)])
    yc, yr = _tc_post(fl(pc), fl(dc), fl(pr), fl(dr), b2,
                      _bd(pcl['lin_W']), _bd(prg['lin_W']), lb)
    return (yc.reshape(N_PAD, HID)[:N_NODES],
            yr.reshape(N_PAD, HID)[:N_NODES])


# speedup vs baseline: 48.3318x; 1.8337x over previous
import functools

import jax
import jax.numpy as jnp
from jax import lax
from jax.experimental import pallas as pl
from jax.experimental.pallas import tpu as pltpu
from jax.experimental.pallas import tpu_sc as plsc

N_NODES = 10000
N_PAD = 10240
HID = 32
NF = N_PAD // 4
N_WORKERS = 32
MC = 128
CHUNKS = 88
E_PAD = N_WORKERS * MC * CHUNKS
N_TILES = 16
ROWS_PER_TILE = N_PAD // N_TILES

_f32 = jnp.float32


def _attn_tabs(h_fl, amat, out_s, out_d, out_m):
    sc = jnp.dot(h_fl, amat, preferred_element_type=_f32)
    a_s = sc[:, 0:4]
    a_d = sc[:, 4:8]
    out_s[...] = a_s
    out_d[...] = a_d
    out_m[...] = jnp.full((1, 128), jnp.max(a_s), _f32)


@functools.partial(
    pl.pallas_call,
    out_shape=(
        jax.ShapeDtypeStruct((NF, 128), _f32),
        jax.ShapeDtypeStruct((NF, 128), _f32),
        jax.ShapeDtypeStruct((NF, 4), _f32),
        jax.ShapeDtypeStruct((NF, 4), _f32),
        jax.ShapeDtypeStruct((1, 128), _f32),
        jax.ShapeDtypeStruct((NF, 4), _f32),
        jax.ShapeDtypeStruct((NF, 4), _f32),
        jax.ShapeDtypeStruct((1, 128), _f32),
    ),
)
def _tc_pre(x_ref, wc_ref, wr_ref, ac_ref, ar_ref, hc_ref, hr_ref,
            asc_ref, adc_ref, mc_ref, asr_ref, adr_ref, mr_ref):
    x = x_ref[...]
    hc = jnp.dot(x, wc_ref[...], preferred_element_type=_f32)
    hr = jnp.dot(x, wr_ref[...], preferred_element_type=_f32)
    hc_ref[...] = hc
    hr_ref[...] = hr
    _attn_tabs(hc, ac_ref[...], asc_ref, adc_ref, mc_ref)
    _attn_tabs(hr, ar_ref[...], asr_ref, adr_ref, mr_ref)


@functools.partial(
    pl.pallas_call,
    out_shape=(
        jax.ShapeDtypeStruct((NF, 128), _f32),
        jax.ShapeDtypeStruct((NF, 128), _f32),
        jax.ShapeDtypeStruct((NF, 4), _f32),
        jax.ShapeDtypeStruct((NF, 4), _f32),
        jax.ShapeDtypeStruct((1, 128), _f32),
        jax.ShapeDtypeStruct((NF, 4), _f32),
        jax.ShapeDtypeStruct((NF, 4), _f32),
        jax.ShapeDtypeStruct((1, 128), _f32),
    ),
)
def _tc_mid(pc_ref, dc_ref, pr_ref, dr_ref, e_ref, b1_ref, w2c_ref, w2r_ref,
            ac_ref, ar_ref,
            h2c_ref, h2r_ref, asc_ref, adc_ref, mc_ref, asr_ref, adr_ref, mr_ref):
    b1 = b1_ref[...]
    e_m = e_ref[...]
    for i, p_ref, d_ref, w_ref, a_ref, h_o, ts, td, tm in (
            (0, pc_ref, dc_ref, w2c_ref, ac_ref, h2c_ref, asc_ref, adc_ref, mc_ref),
            (1, pr_ref, dr_ref, w2r_ref, ar_ref, h2r_ref, asr_ref, adr_ref, mr_ref)):
        p = p_ref[...]
        d = d_ref[...]
        den = jnp.dot(d[0] + d[1], e_m, preferred_element_type=_f32) + 1e-16
        g = jnp.maximum((p[0] + p[1]) / den + b1[i:i + 1], 0.0)
        h2 = jnp.dot(g, w_ref[...], preferred_element_type=_f32)
        h_o[...] = h2
        _attn_tabs(h2, a_ref[...], ts, td, tm)


@functools.partial(
    pl.pallas_call,
    out_shape=(
        jax.ShapeDtypeStruct((NF, 128), _f32),
        jax.ShapeDtypeStruct((NF, 128), _f32),
    ),
)
def _tc_post(pc_ref, dc_ref, pr_ref, dr_ref, e_ref, b2_ref, lwc_ref, lwr_ref,
             lb_ref, yc_ref, yr_ref):
    b2 = b2_ref[...]
    lb = lb_ref[...]
    e_m = e_ref[...]
    for i, p_ref, d_ref, w_ref, y_ref in ((0, pc_ref, dc_ref, lwc_ref, yc_ref),
                                          (1, pr_ref, dr_ref, lwr_ref, yr_ref)):
        p = p_ref[...]
        d = d_ref[...]
        den = jnp.dot(d[0] + d[1], e_m, preferred_element_type=_f32) + 1e-16
        g = jnp.maximum((p[0] + p[1]) / den + b2[i:i + 1], 0.0)
        y = jnp.dot(g, w_ref[...], preferred_element_type=_f32) + lb[i:i + 1]
        if i == 0:
            y = jax.nn.sigmoid(y)
        y_ref[...] = y


_SC_OUT_TYPE = (
    jax.ShapeDtypeStruct((2, N_PAD, HID), _f32),
    jax.ShapeDtypeStruct((2, N_PAD), _f32),
    jax.ShapeDtypeStruct((2, N_PAD, HID), _f32),
    jax.ShapeDtypeStruct((2, N_PAD), _f32),
)

_SET_SCRATCH = [
    pltpu.VMEM((MC,), _f32),
    pltpu.VMEM((MC,), _f32),
    pltpu.VMEM((MC,), _f32),
    pltpu.VMEM((MC,), _f32),
    pltpu.VMEM((MC, HID), _f32),
    pltpu.VMEM((MC, HID), _f32),
    pltpu.VMEM((MC,), _f32),
    pltpu.VMEM((MC,), _f32),
    pltpu.SemaphoreType.DMA,
    pltpu.SemaphoreType.DMA,
]

_SC_SCRATCH = [
    pltpu.VMEM((CHUNKS, MC), jnp.int32),
    pltpu.VMEM((CHUNKS, MC), jnp.int32),
    pltpu.VMEM((2, 128), _f32),
] + _SET_SCRATCH + _SET_SCRATCH + [
    pltpu.VMEM_SHARED((N_PAD, HID), _f32),
    pltpu.VMEM_SHARED((N_PAD,), _f32),
    pltpu.VMEM_SHARED((N_PAD, HID), _f32),
    pltpu.VMEM_SHARED((N_PAD,), _f32),
]

_G_DNUMS = lax.GatherDimensionNumbers(
    offset_dims=(), collapsed_slice_dims=(0,), start_index_map=(0,))


def _sc_edge_body(srcr, dstr, hc, asc, adc, hr, asr, adr, m2, z2, z1,
                  pc, dc, pr, dr,
                  srcv, dstv, mv, *rest):
    set0, set1 = rest[0:10], rest[10:20]
    accc, denc, accr, denr = rest[20:24]
    sets = (set0, set1)
    cid = lax.axis_index("c")
    sid = lax.axis_index("s")
    wid = sid * 2 + cid
    r0 = sid * ROWS_PER_TILE

    pltpu.sync_copy(z2.at[pl.ds(r0, ROWS_PER_TILE)], accc.at[pl.ds(r0, ROWS_PER_TILE)])
    pltpu.sync_copy(z2.at[pl.ds(r0, ROWS_PER_TILE)], accr.at[pl.ds(r0, ROWS_PER_TILE)])
    pltpu.sync_copy(z1.at[pl.ds(r0, ROWS_PER_TILE)], denc.at[pl.ds(r0, ROWS_PER_TILE)])
    pltpu.sync_copy(z1.at[pl.ds(r0, ROWS_PER_TILE)], denr.at[pl.ds(r0, ROWS_PER_TILE)])
    plsc.subcore_barrier()

    pltpu.sync_copy(srcr.at[pl.ds(wid * CHUNKS, CHUNKS)], srcv)
    pltpu.sync_copy(dstr.at[pl.ds(wid * CHUNKS, CHUNKS)], dstv)
    pltpu.sync_copy(m2, mv)
    for s in sets:
        pltpu.sync_copy(z1.at[pl.ds(0, MC)], s[6])
        pltpu.sync_copy(z1.at[pl.ds(0, MC)], s[7])

    def _fire_gathers(j, s):
        si = srcv.at[j]
        di = dstv.at[j]
        sem = s[8]
        pltpu.async_copy(asc.at[si], s[0], sem)
        pltpu.async_copy(adc.at[di], s[1], sem)
        pltpu.async_copy(asr.at[si], s[2], sem)
        pltpu.async_copy(adr.at[di], s[3], sem)
        pltpu.async_copy(hc.at[si], s[4], sem)
        pltpu.async_copy(hr.at[si], s[5], sem)

    def _drain_gathers(j, s):
        si = srcv.at[j]
        di = dstv.at[j]
        sem = s[8]
        pltpu.make_async_copy(asc.at[si], s[0], sem).wait()
        pltpu.make_async_copy(adc.at[di], s[1], sem).wait()
        pltpu.make_async_copy(asr.at[si], s[2], sem).wait()
        pltpu.make_async_copy(adr.at[di], s[3], sem).wait()
        pltpu.make_async_copy(hc.at[si], s[4], sem).wait()
        pltpu.make_async_copy(hr.at[si], s[5], sem).wait()

    def _fire_scatters(j, s):
        di = dstv.at[j]
        sem = s[9]
        pltpu.async_copy(s[4], accc.at[di], sem, add=True)
        pltpu.async_copy(s[5], accr.at[di], sem, add=True)
        pltpu.async_copy(s[6], denc.at[di], sem, add=True)
        pltpu.async_copy(s[7], denr.at[di], sem, add=True)

    def _drain_scatters(s):
        di = dstv.at[0]
        sem = s[9]
        pltpu.make_async_copy(s[4], accc.at[di], sem).wait()
        pltpu.make_async_copy(s[5], accr.at[di], sem).wait()
        pltpu.make_async_copy(s[6], denc.at[di], sem).wait()
        pltpu.make_async_copy(s[7], denr.at[di], sem).wait()

    def _compute(j, s, bi):
        ase, ade, hrow, wv = s[0 + bi], s[1 + bi], s[4 + bi // 2], s[6 + bi // 2]

        def _grp(kk, c2):
            s_ = pl.ds(kk * 16, 16)
            av = ase[s_]
            dv = ade[s_]
            z = av + dv
            z = jnp.maximum(z, 0.2 * z)
            t = mv[bi // 2, pl.ds(0, 16)] + dv
            z = z - jnp.maximum(t, 0.2 * t)
            w = jnp.exp(z)
            wv[s_] = w + wv[s_] * 0.0
            for i in range(16):
                r = kk * 16 + i
                wb = lax.gather(w, jnp.full((16, 1), i, jnp.int32), _G_DNUMS,
                                (1,), mode=lax.GatherScatterMode.PROMISE_IN_BOUNDS)
                hrow[r, pl.ds(0, 16)] = hrow[r, pl.ds(0, 16)] * wb
                hrow[r, pl.ds(16, 16)] = hrow[r, pl.ds(16, 16)] * wb
            return c2

        lax.fori_loop(0, MC // 16, _grp, 0)

    _fire_gathers(0, sets[0])

    def _pair(jj, carry):
        for b in (0, 1):
            j = 2 * jj + b
            cur = sets[b]
            nxt = sets[1 - b]

            @pl.when(j > 0)
            def _():
                _drain_scatters(nxt)

            @pl.when(j + 1 < CHUNKS)
            def _():
                _fire_gathers(j + 1, nxt)

            _drain_gathers(j, cur)
            _compute(j, cur, 0)
            _compute(j, cur, 2)
            _fire_scatters(j, cur)
        return carry

    lax.fori_loop(0, CHUNKS // 2, _pair, 0)
    _drain_scatters(sets[1])
    plsc.subcore_barrier()

    pltpu.sync_copy(accc.at[pl.ds(r0, ROWS_PER_TILE)],
                    pc.at[cid, pl.ds(r0, ROWS_PER_TILE)])
    pltpu.sync_copy(accr.at[pl.ds(r0, ROWS_PER_TILE)],
                    pr.at[cid, pl.ds(r0, ROWS_PER_TILE)])
    pltpu.sync_copy(denc.at[pl.ds(r0, ROWS_PER_TILE)],
                    dc.at[cid, pl.ds(r0, ROWS_PER_TILE)])
    pltpu.sync_copy(denr.at[pl.ds(r0, ROWS_PER_TILE)],
                    dr.at[cid, pl.ds(r0, ROWS_PER_TILE)])


@functools.cache
def _sc_edge():
    mesh = plsc.VectorSubcoreMesh(core_axis_name="c", subcore_axis_name="s",
                                  num_cores=2, num_subcores=N_TILES)
    return pl.kernel(_sc_edge_body, out_type=_SC_OUT_TYPE, mesh=mesh,
                     scratch_types=_SC_SCRATCH,
                     compiler_params=pltpu.CompilerParams(use_tc_tiling_on_sc=False))


def _bd(w):
    return jnp.kron(jnp.eye(4, dtype=_f32), w)


def _amat(att_src, att_dst):
    e4 = jnp.eye(4, dtype=_f32)
    return jnp.concatenate([jnp.kron(e4, att_src[:, None]),
                            jnp.kron(e4, att_dst[:, None])], axis=1)


def kernel(x, edge_index, params):
    pcl, prg = params['cls'], params['reg']

    xp = jnp.pad(x, ((0, N_PAD - N_NODES), (0, 0))).reshape(NF, 512)
    loops = jnp.arange(N_NODES, dtype=jnp.int32)
    n_fill = E_PAD - N_NODES - edge_index.shape[1]
    fill = jnp.full((n_fill,), N_NODES, jnp.int32)
    srcr = jnp.concatenate([edge_index[0], loops, fill]).reshape(E_PAD // MC, MC)
    dstr = jnp.concatenate([edge_index[1], loops, fill]).reshape(E_PAD // MC, MC)
    z2 = jnp.zeros((N_PAD, HID), _f32)
    z1 = jnp.zeros((N_PAD,), _f32)
    e_m = jnp.kron(jnp.eye(4, dtype=_f32), jnp.ones((1, HID), _f32))
    sq = lambda t: t.reshape(N_PAD)
    fl = lambda t: t.reshape(2, NF, 128)
    fd = lambda t: t.reshape(2, NF, 4)

    w1c = jnp.kron(jnp.eye(4, dtype=_f32), pcl['W1'])
    w1r = jnp.kron(jnp.eye(4, dtype=_f32), prg['W1'])
    a1c = _amat(pcl['att_src1'], pcl['att_dst1'])
    a1r = _amat(prg['att_src1'], prg['att_dst1'])
    hc, hr, asc, adc, mc_, asr, adr, mr_ = _tc_pre(xp, w1c, w1r, a1c, a1r)
    m2 = jnp.concatenate([mc_, mr_], axis=0)

    pc, dc, pr, dr = _sc_edge()(srcr, dstr,
                                hc.reshape(N_PAD, HID), sq(asc), sq(adc),
                                hr.reshape(N_PAD, HID), sq(asr), sq(adr),
                                m2, z2, z1)

    b1 = jnp.stack([jnp.tile(pcl['b1'], 4), jnp.tile(prg['b1'], 4)])
    h2c, h2r, asc, adc, mc_, asr, adr, mr_ = _tc_mid(
        fl(pc), fd(dc), fl(pr), fd(dr), e_m, b1, _bd(pcl['W2']), _bd(prg['W2']),
        _amat(pcl['att_src2'], pcl['att_dst2']),
        _amat(prg['att_src2'], prg['att_dst2']))
    m2 = jnp.concatenate([mc_, mr_], axis=0)

    pc, dc, pr, dr = _sc_edge()(srcr, dstr,
                                h2c.reshape(N_PAD, HID), sq(asc), sq(adc),
                                h2r.reshape(N_PAD, HID), sq(asr), sq(adr),
                                m2, z2, z1)

    b2 = jnp.stack([jnp.tile(pcl['b2'], 4), jnp.tile(prg['b2'], 4)])
    lb = jnp.stack([jnp.tile(pcl['lin_b'], 4), jnp.tile(prg['lin_b'], 4)])
    yc, yr = _tc_post(fl(pc), fd(dc), fl(pr), fd(dr), e_m, b2,
                      _bd(pcl['lin_W']), _bd(prg['lin_W']), lb)
    return (yc.reshape(N_PAD, HID)[:N_NODES],
            yr.reshape(N_PAD, HID)[:N_NODES])


# speedup vs baseline: 84.0818x; 1.7397x over previous
import functools

import jax
import jax.numpy as jnp
from jax import lax
from jax.experimental import pallas as pl
from jax.experimental.pallas import tpu as pltpu
from jax.experimental.pallas import tpu_sc as plsc

N_NODES = 10000
N_PAD = 10240
HID = 32
NF = N_PAD // 4
N_WORKERS = 32
MC = 128
CHUNKS = 80
E_PAD = N_WORKERS * MC * CHUNKS
N_TILES = 16
ROWS_PER_TILE = N_PAD // N_TILES

_f32 = jnp.float32


def _attn_tabs(h_fl, amat, out_s, out_d, out_m):
    sc = jnp.dot(h_fl, amat, preferred_element_type=_f32)
    a_s = sc[:, 0:4]
    a_d = sc[:, 4:8]
    out_s[...] = a_s
    out_d[...] = a_d
    out_m[...] = jnp.full((1, 128), jnp.max(a_s), _f32)


@functools.partial(
    pl.pallas_call,
    out_shape=(
        jax.ShapeDtypeStruct((NF, 128), _f32),
        jax.ShapeDtypeStruct((NF, 128), _f32),
        jax.ShapeDtypeStruct((NF, 4), _f32),
        jax.ShapeDtypeStruct((NF, 4), _f32),
        jax.ShapeDtypeStruct((1, 128), _f32),
        jax.ShapeDtypeStruct((NF, 4), _f32),
        jax.ShapeDtypeStruct((NF, 4), _f32),
        jax.ShapeDtypeStruct((1, 128), _f32),
    ),
)
def _tc_pre(x_ref, wc_ref, wr_ref, ac_ref, ar_ref, hc_ref, hr_ref,
            asc_ref, adc_ref, mc_ref, asr_ref, adr_ref, mr_ref):
    x = x_ref[...]
    hc = jnp.dot(x, wc_ref[...], preferred_element_type=_f32)
    hr = jnp.dot(x, wr_ref[...], preferred_element_type=_f32)
    hc_ref[...] = hc
    hr_ref[...] = hr
    _attn_tabs(hc, ac_ref[...], asc_ref, adc_ref, mc_ref)
    _attn_tabs(hr, ar_ref[...], asr_ref, adr_ref, mr_ref)


@functools.partial(
    pl.pallas_call,
    out_shape=(
        jax.ShapeDtypeStruct((NF, 128), _f32),
        jax.ShapeDtypeStruct((NF, 128), _f32),
        jax.ShapeDtypeStruct((NF, 4), _f32),
        jax.ShapeDtypeStruct((NF, 4), _f32),
        jax.ShapeDtypeStruct((1, 128), _f32),
        jax.ShapeDtypeStruct((NF, 4), _f32),
        jax.ShapeDtypeStruct((NF, 4), _f32),
        jax.ShapeDtypeStruct((1, 128), _f32),
    ),
)
def _tc_mid(pc_ref, dc_ref, pr_ref, dr_ref, e_ref, b1_ref, w2c_ref, w2r_ref,
            ac_ref, ar_ref, hc_ref, hr_ref,
            sc_ref, dcc_ref, mcc_ref, sr_ref, drr_ref, mrr_ref,
            h2c_ref, h2r_ref, asc_ref, adc_ref, mc_ref, asr_ref, adr_ref, mr_ref):
    b1 = b1_ref[...]
    e_m = e_ref[...]
    for i, p_ref, d_ref, w_ref, a_ref, h_ref, s_ref, dd_ref, m_ref, h_o, ts, td, tm in (
            (0, pc_ref, dc_ref, w2c_ref, ac_ref, hc_ref, sc_ref, dcc_ref, mcc_ref,
             h2c_ref, asc_ref, adc_ref, mc_ref),
            (1, pr_ref, dr_ref, w2r_ref, ar_ref, hr_ref, sr_ref, drr_ref, mrr_ref,
             h2r_ref, asr_ref, adr_ref, mr_ref)):
        p = p_ref[...]
        d = d_ref[...]
        a_s = s_ref[...]
        a_d = dd_ref[...]
        t = m_ref[0:1, 0:4] + a_d
        b = jnp.maximum(t, 0.2 * t)
        zs = a_s + a_d
        w4 = jnp.exp(jnp.maximum(zs, 0.2 * zs) - b)
        den = jnp.dot(d[0] + d[1] + w4, e_m, preferred_element_type=_f32) + 1e-16
        num = p[0] + p[1] + jnp.dot(w4, e_m, preferred_element_type=_f32) * h_ref[...]
        g = jnp.maximum(num / den + b1[i:i + 1], 0.0)
        h2 = jnp.dot(g, w_ref[...], preferred_element_type=_f32)
        h_o[...] = h2
        _attn_tabs(h2, a_ref[...], ts, td, tm)


@functools.partial(
    pl.pallas_call,
    out_shape=(
        jax.ShapeDtypeStruct((NF, 128), _f32),
        jax.ShapeDtypeStruct((NF, 128), _f32),
    ),
)
def _tc_post(pc_ref, dc_ref, pr_ref, dr_ref, e_ref, b2_ref, lwc_ref, lwr_ref,
             lb_ref, hc_ref, hr_ref,
             sc_ref, dcc_ref, mcc_ref, sr_ref, drr_ref, mrr_ref,
             yc_ref, yr_ref):
    b2 = b2_ref[...]
    lb = lb_ref[...]
    e_m = e_ref[...]
    for i, p_ref, d_ref, w_ref, h_ref, s_ref, dd_ref, m_ref, y_ref in (
            (0, pc_ref, dc_ref, lwc_ref, hc_ref, sc_ref, dcc_ref, mcc_ref, yc_ref),
            (1, pr_ref, dr_ref, lwr_ref, hr_ref, sr_ref, drr_ref, mrr_ref, yr_ref)):
        p = p_ref[...]
        d = d_ref[...]
        a_s = s_ref[...]
        a_d = dd_ref[...]
        t = m_ref[0:1, 0:4] + a_d
        b = jnp.maximum(t, 0.2 * t)
        zs = a_s + a_d
        w4 = jnp.exp(jnp.maximum(zs, 0.2 * zs) - b)
        den = jnp.dot(d[0] + d[1] + w4, e_m, preferred_element_type=_f32) + 1e-16
        num = p[0] + p[1] + jnp.dot(w4, e_m, preferred_element_type=_f32) * h_ref[...]
        g = jnp.maximum(num / den + b2[i:i + 1], 0.0)
        y = jnp.dot(g, w_ref[...], preferred_element_type=_f32) + lb[i:i + 1]
        if i == 0:
            y = jax.nn.sigmoid(y)
        y_ref[...] = y


_SC_OUT_TYPE = (
    jax.ShapeDtypeStruct((2, N_PAD, HID), _f32),
    jax.ShapeDtypeStruct((2, N_PAD), _f32),
    jax.ShapeDtypeStruct((2, N_PAD, HID), _f32),
    jax.ShapeDtypeStruct((2, N_PAD), _f32),
)

_SET_SCRATCH = [
    pltpu.VMEM((MC,), _f32),
    pltpu.VMEM((MC,), _f32),
    pltpu.VMEM((MC,), _f32),
    pltpu.VMEM((MC,), _f32),
    pltpu.VMEM((MC, HID), _f32),
    pltpu.VMEM((MC, HID), _f32),
    pltpu.VMEM((MC,), _f32),
    pltpu.VMEM((MC,), _f32),
    pltpu.SemaphoreType.DMA,
    pltpu.SemaphoreType.DMA,
]

_SC_SCRATCH = [
    pltpu.VMEM((CHUNKS, MC), jnp.int32),
    pltpu.VMEM((CHUNKS, MC), jnp.int32),
    pltpu.VMEM((2, 128), _f32),
] + _SET_SCRATCH + _SET_SCRATCH + [
    pltpu.VMEM_SHARED((N_PAD, HID), _f32),
    pltpu.VMEM_SHARED((N_PAD,), _f32),
    pltpu.VMEM_SHARED((N_PAD, HID), _f32),
    pltpu.VMEM_SHARED((N_PAD,), _f32),
]

_G_DNUMS = lax.GatherDimensionNumbers(
    offset_dims=(), collapsed_slice_dims=(0,), start_index_map=(0,))


def _sc_edge_body(srcr, dstr, hc, asc, adc, hr, asr, adr, m2, z2, z1,
                  pc, dc, pr, dr,
                  srcv, dstv, mv, *rest):
    set0, set1 = rest[0:10], rest[10:20]
    accc, denc, accr, denr = rest[20:24]
    sets = (set0, set1)
    cid = lax.axis_index("c")
    sid = lax.axis_index("s")
    wid = sid * 2 + cid
    r0 = sid * ROWS_PER_TILE

    pltpu.sync_copy(z2.at[pl.ds(r0, ROWS_PER_TILE)], accc.at[pl.ds(r0, ROWS_PER_TILE)])
    pltpu.sync_copy(z2.at[pl.ds(r0, ROWS_PER_TILE)], accr.at[pl.ds(r0, ROWS_PER_TILE)])
    pltpu.sync_copy(z1.at[pl.ds(r0, ROWS_PER_TILE)], denc.at[pl.ds(r0, ROWS_PER_TILE)])
    pltpu.sync_copy(z1.at[pl.ds(r0, ROWS_PER_TILE)], denr.at[pl.ds(r0, ROWS_PER_TILE)])
    plsc.subcore_barrier()

    pltpu.sync_copy(srcr.at[pl.ds(wid * CHUNKS, CHUNKS)], srcv)
    pltpu.sync_copy(dstr.at[pl.ds(wid * CHUNKS, CHUNKS)], dstv)
    pltpu.sync_copy(m2, mv)
    for s in sets:
        pltpu.sync_copy(z1.at[pl.ds(0, MC)], s[6])
        pltpu.sync_copy(z1.at[pl.ds(0, MC)], s[7])

    def _fire_gathers(j, s):
        si = srcv.at[j]
        di = dstv.at[j]
        sem = s[8]
        pltpu.async_copy(asc.at[si], s[0], sem)
        pltpu.async_copy(adc.at[di], s[1], sem)
        pltpu.async_copy(asr.at[si], s[2], sem)
        pltpu.async_copy(adr.at[di], s[3], sem)
        pltpu.async_copy(hc.at[si], s[4], sem)
        pltpu.async_copy(hr.at[si], s[5], sem)

    def _drain_gathers(j, s):
        si = srcv.at[j]
        di = dstv.at[j]
        sem = s[8]
        pltpu.make_async_copy(asc.at[si], s[0], sem).wait()
        pltpu.make_async_copy(adc.at[di], s[1], sem).wait()
        pltpu.make_async_copy(asr.at[si], s[2], sem).wait()
        pltpu.make_async_copy(adr.at[di], s[3], sem).wait()
        pltpu.make_async_copy(hc.at[si], s[4], sem).wait()
        pltpu.make_async_copy(hr.at[si], s[5], sem).wait()

    def _fire_scatters(j, s):
        di = dstv.at[j]
        sem = s[9]
        pltpu.async_copy(s[4], accc.at[di], sem, add=True)
        pltpu.async_copy(s[5], accr.at[di], sem, add=True)
        pltpu.async_copy(s[6], denc.at[di], sem, add=True)
        pltpu.async_copy(s[7], denr.at[di], sem, add=True)

    def _drain_scatters(s):
        di = dstv.at[0]
        sem = s[9]
        pltpu.make_async_copy(s[4], accc.at[di], sem).wait()
        pltpu.make_async_copy(s[5], accr.at[di], sem).wait()
        pltpu.make_async_copy(s[6], denc.at[di], sem).wait()
        pltpu.make_async_copy(s[7], denr.at[di], sem).wait()

    def _compute(j, s, bi):
        ase, ade, hrow, wv = s[0 + bi], s[1 + bi], s[4 + bi // 2], s[6 + bi // 2]

        def _grp(kk, c2):
            s_ = pl.ds(kk * 16, 16)
            av = ase[s_]
            dv = ade[s_]
            z = av + dv
            z = jnp.maximum(z, 0.2 * z)
            t = mv[bi // 2, pl.ds(0, 16)] + dv
            z = z - jnp.maximum(t, 0.2 * t)
            w = jnp.exp(z)
            wv[s_] = w + wv[s_] * 0.0
            for i in range(16):
                r = kk * 16 + i
                wb = lax.gather(w, jnp.full((16, 1), i, jnp.int32), _G_DNUMS,
                                (1,), mode=lax.GatherScatterMode.PROMISE_IN_BOUNDS)
                hrow[r, pl.ds(0, 16)] = hrow[r, pl.ds(0, 16)] * wb
                hrow[r, pl.ds(16, 16)] = hrow[r, pl.ds(16, 16)] * wb
            return c2

        lax.fori_loop(0, MC // 16, _grp, 0)

    _fire_gathers(0, sets[0])

    def _pair(jj, carry):
        for b in (0, 1):
            j = 2 * jj + b
            cur = sets[b]
            nxt = sets[1 - b]

            @pl.when(j > 0)
            def _():
                _drain_scatters(nxt)

            @pl.when(j + 1 < CHUNKS)
            def _():
                _fire_gathers(j + 1, nxt)

            _drain_gathers(j, cur)
            _compute(j, cur, 0)
            _compute(j, cur, 2)
            _fire_scatters(j, cur)
        return carry

    lax.fori_loop(0, CHUNKS // 2, _pair, 0)
    _drain_scatters(sets[1])
    plsc.subcore_barrier()

    pltpu.sync_copy(accc.at[pl.ds(r0, ROWS_PER_TILE)],
                    pc.at[cid, pl.ds(r0, ROWS_PER_TILE)])
    pltpu.sync_copy(accr.at[pl.ds(r0, ROWS_PER_TILE)],
                    pr.at[cid, pl.ds(r0, ROWS_PER_TILE)])
    pltpu.sync_copy(denc.at[pl.ds(r0, ROWS_PER_TILE)],
                    dc.at[cid, pl.ds(r0, ROWS_PER_TILE)])
    pltpu.sync_copy(denr.at[pl.ds(r0, ROWS_PER_TILE)],
                    dr.at[cid, pl.ds(r0, ROWS_PER_TILE)])


@functools.cache
def _sc_edge():
    mesh = plsc.VectorSubcoreMesh(core_axis_name="c", subcore_axis_name="s",
                                  num_cores=2, num_subcores=N_TILES)
    return pl.kernel(_sc_edge_body, out_type=_SC_OUT_TYPE, mesh=mesh,
                     scratch_types=_SC_SCRATCH,
                     compiler_params=pltpu.CompilerParams(use_tc_tiling_on_sc=False))


def _bd(w):
    return jnp.kron(jnp.eye(4, dtype=_f32), w)


def _amat(att_src, att_dst):
    e4 = jnp.eye(4, dtype=_f32)
    return jnp.concatenate([jnp.kron(e4, att_src[:, None]),
                            jnp.kron(e4, att_dst[:, None])], axis=1)


def kernel(x, edge_index, params):
    pcl, prg = params['cls'], params['reg']

    xp = jnp.pad(x, ((0, N_PAD - N_NODES), (0, 0))).reshape(NF, 512)
    n_fill = E_PAD - edge_index.shape[1]
    fill = jnp.full((n_fill,), N_NODES, jnp.int32)
    srcr = jnp.concatenate([edge_index[0], fill]).reshape(E_PAD // MC, MC)
    dstr = jnp.concatenate([edge_index[1], fill]).reshape(E_PAD // MC, MC)
    z2 = jnp.zeros((N_PAD, HID), _f32)
    z1 = jnp.zeros((N_PAD,), _f32)
    e_m = jnp.kron(jnp.eye(4, dtype=_f32), jnp.ones((1, HID), _f32))
    sq = lambda t: t.reshape(N_PAD)
    fl = lambda t: t.reshape(2, NF, 128)
    fd = lambda t: t.reshape(2, NF, 4)

    w1c = jnp.kron(jnp.eye(4, dtype=_f32), pcl['W1'])
    w1r = jnp.kron(jnp.eye(4, dtype=_f32), prg['W1'])
    a1c = _amat(pcl['att_src1'], pcl['att_dst1'])
    a1r = _amat(prg['att_src1'], prg['att_dst1'])
    hc, hr, asc, adc, mc_, asr, adr, mr_ = _tc_pre(xp, w1c, w1r, a1c, a1r)
    m2 = jnp.concatenate([mc_, mr_], axis=0)

    pc, dc, pr, dr = _sc_edge()(srcr, dstr,
                                hc.reshape(N_PAD, HID), sq(asc), sq(adc),
                                hr.reshape(N_PAD, HID), sq(asr), sq(adr),
                                m2, z2, z1)

    b1 = jnp.stack([jnp.tile(pcl['b1'], 4), jnp.tile(prg['b1'], 4)])
    h2c, h2r, asc2, adc2, mc2, asr2, adr2, mr2 = _tc_mid(
        fl(pc), fd(dc), fl(pr), fd(dr), e_m, b1, _bd(pcl['W2']), _bd(prg['W2']),
        _amat(pcl['att_src2'], pcl['att_dst2']),
        _amat(prg['att_src2'], prg['att_dst2']),
        hc, hr, asc, adc, mc_, asr, adr, mr_)
    m2 = jnp.concatenate([mc2, mr2], axis=0)

    pc, dc, pr, dr = _sc_edge()(srcr, dstr,
                                h2c.reshape(N_PAD, HID), sq(asc2), sq(adc2),
                                h2r.reshape(N_PAD, HID), sq(asr2), sq(adr2),
                                m2, z2, z1)

    b2 = jnp.stack([jnp.tile(pcl['b2'], 4), jnp.tile(prg['b2'], 4)])
    lb = jnp.stack([jnp.tile(pcl['lin_b'], 4), jnp.tile(prg['lin_b'], 4)])
    yc, yr = _tc_post(fl(pc), fd(dc), fl(pr), fd(dr), e_m, b2,
                      _bd(pcl['lin_W']), _bd(prg['lin_W']), lb,
                      h2c, h2r, asc2, adc2, mc2, asr2, adr2, mr2)
    return (yc.reshape(N_PAD, HID)[:N_NODES],
            yr.reshape(N_PAD, HID)[:N_NODES])


# speedup vs baseline: 84.1261x; 1.0005x over previous
import functools

import jax
import jax.numpy as jnp
from jax import lax
from jax.experimental import pallas as pl
from jax.experimental.pallas import tpu as pltpu
from jax.experimental.pallas import tpu_sc as plsc

N_NODES = 10000
N_PAD = 10240
HID = 32
NF = N_PAD // 4
N_WORKERS = 32
MC = 128
CHUNKS = 80
E_PAD = N_WORKERS * MC * CHUNKS
N_TILES = 16
ROWS_PER_TILE = N_PAD // N_TILES

_f32 = jnp.float32


def _attn_tabs(h_fl, amat, out_s, out_d, out_m):
    sc = jnp.dot(h_fl, amat, preferred_element_type=_f32)
    a_s = sc[:, 0:4]
    a_d = sc[:, 4:8]
    out_s[...] = a_s
    out_d[...] = a_d
    out_m[...] = jnp.full((1, 128), jnp.max(a_s), _f32)


@functools.partial(
    pl.pallas_call,
    out_shape=(
        jax.ShapeDtypeStruct((NF, 128), _f32),
        jax.ShapeDtypeStruct((NF, 128), _f32),
        jax.ShapeDtypeStruct((NF, 4), _f32),
        jax.ShapeDtypeStruct((NF, 4), _f32),
        jax.ShapeDtypeStruct((1, 128), _f32),
        jax.ShapeDtypeStruct((NF, 4), _f32),
        jax.ShapeDtypeStruct((NF, 4), _f32),
        jax.ShapeDtypeStruct((1, 128), _f32),
    ),
)
def _tc_pre(x_ref, wc_ref, wr_ref, ac_ref, ar_ref, hc_ref, hr_ref,
            asc_ref, adc_ref, mc_ref, asr_ref, adr_ref, mr_ref):
    x = x_ref[...]
    hc = jnp.dot(x, wc_ref[...], preferred_element_type=_f32)
    hr = jnp.dot(x, wr_ref[...], preferred_element_type=_f32)
    hc_ref[...] = hc
    hr_ref[...] = hr
    _attn_tabs(hc, ac_ref[...], asc_ref, adc_ref, mc_ref)
    _attn_tabs(hr, ar_ref[...], asr_ref, adr_ref, mr_ref)


@functools.partial(
    pl.pallas_call,
    out_shape=(
        jax.ShapeDtypeStruct((NF, 128), _f32),
        jax.ShapeDtypeStruct((NF, 128), _f32),
        jax.ShapeDtypeStruct((NF, 4), _f32),
        jax.ShapeDtypeStruct((NF, 4), _f32),
        jax.ShapeDtypeStruct((1, 128), _f32),
        jax.ShapeDtypeStruct((NF, 4), _f32),
        jax.ShapeDtypeStruct((NF, 4), _f32),
        jax.ShapeDtypeStruct((1, 128), _f32),
    ),
)
def _tc_mid(pc_ref, dc_ref, pr_ref, dr_ref, e_ref, b1_ref, w2c_ref, w2r_ref,
            ac_ref, ar_ref, hc_ref, hr_ref,
            sc_ref, dcc_ref, mcc_ref, sr_ref, drr_ref, mrr_ref,
            h2c_ref, h2r_ref, asc_ref, adc_ref, mc_ref, asr_ref, adr_ref, mr_ref):
    b1 = b1_ref[...]
    e_m = e_ref[...]
    for i, p_ref, d_ref, w_ref, a_ref, h_ref, s_ref, dd_ref, m_ref, h_o, ts, td, tm in (
            (0, pc_ref, dc_ref, w2c_ref, ac_ref, hc_ref, sc_ref, dcc_ref, mcc_ref,
             h2c_ref, asc_ref, adc_ref, mc_ref),
            (1, pr_ref, dr_ref, w2r_ref, ar_ref, hr_ref, sr_ref, drr_ref, mrr_ref,
             h2r_ref, asr_ref, adr_ref, mr_ref)):
        p = p_ref[...]
        d = d_ref[...]
        a_s = s_ref[...]
        a_d = dd_ref[...]
        t = m_ref[0:1, 0:4] + a_d
        b = jnp.maximum(t, 0.2 * t)
        zs = a_s + a_d
        w4 = jnp.exp(jnp.maximum(zs, 0.2 * zs) - b)
        den = jnp.dot(d[0] + d[1] + w4, e_m, preferred_element_type=_f32) + 1e-16
        num = p[0] + p[1] + jnp.dot(w4, e_m, preferred_element_type=_f32) * h_ref[...]
        g = jnp.maximum(num / den + b1[i:i + 1], 0.0)
        h2 = jnp.dot(g, w_ref[...], preferred_element_type=_f32)
        h_o[...] = h2
        _attn_tabs(h2, a_ref[...], ts, td, tm)


@functools.partial(
    pl.pallas_call,
    out_shape=(
        jax.ShapeDtypeStruct((NF, 128), _f32),
        jax.ShapeDtypeStruct((NF, 128), _f32),
    ),
)
def _tc_post(pc_ref, dc_ref, pr_ref, dr_ref, e_ref, b2_ref, lwc_ref, lwr_ref,
             lb_ref, hc_ref, hr_ref,
             sc_ref, dcc_ref, mcc_ref, sr_ref, drr_ref, mrr_ref,
             yc_ref, yr_ref):
    b2 = b2_ref[...]
    lb = lb_ref[...]
    e_m = e_ref[...]
    for i, p_ref, d_ref, w_ref, h_ref, s_ref, dd_ref, m_ref, y_ref in (
            (0, pc_ref, dc_ref, lwc_ref, hc_ref, sc_ref, dcc_ref, mcc_ref, yc_ref),
            (1, pr_ref, dr_ref, lwr_ref, hr_ref, sr_ref, drr_ref, mrr_ref, yr_ref)):
        p = p_ref[...]
        d = d_ref[...]
        a_s = s_ref[...]
        a_d = dd_ref[...]
        t = m_ref[0:1, 0:4] + a_d
        b = jnp.maximum(t, 0.2 * t)
        zs = a_s + a_d
        w4 = jnp.exp(jnp.maximum(zs, 0.2 * zs) - b)
        den = jnp.dot(d[0] + d[1] + w4, e_m, preferred_element_type=_f32) + 1e-16
        num = p[0] + p[1] + jnp.dot(w4, e_m, preferred_element_type=_f32) * h_ref[...]
        g = jnp.maximum(num / den + b2[i:i + 1], 0.0)
        y = jnp.dot(g, w_ref[...], preferred_element_type=_f32) + lb[i:i + 1]
        if i == 0:
            y = jax.nn.sigmoid(y)
        y_ref[...] = y


_SC_OUT_TYPE = (
    jax.ShapeDtypeStruct((2, N_PAD, HID), _f32),
    jax.ShapeDtypeStruct((2, N_PAD), _f32),
    jax.ShapeDtypeStruct((2, N_PAD, HID), _f32),
    jax.ShapeDtypeStruct((2, N_PAD), _f32),
)

_SET_SCRATCH = [
    pltpu.VMEM((MC,), _f32),
    pltpu.VMEM((MC,), _f32),
    pltpu.VMEM((MC,), _f32),
    pltpu.VMEM((MC,), _f32),
    pltpu.VMEM((MC, HID), _f32),
    pltpu.VMEM((MC, HID), _f32),
    pltpu.VMEM((MC,), _f32),
    pltpu.VMEM((MC,), _f32),
    pltpu.SemaphoreType.DMA,
    pltpu.SemaphoreType.DMA,
]

_SC_SCRATCH = [
    pltpu.VMEM((CHUNKS, MC), jnp.int32),
    pltpu.VMEM((CHUNKS, MC), jnp.int32),
    pltpu.VMEM((2, 128), _f32),
] + _SET_SCRATCH + _SET_SCRATCH + [
    pltpu.VMEM_SHARED((N_PAD, HID), _f32),
    pltpu.VMEM_SHARED((N_PAD,), _f32),
    pltpu.VMEM_SHARED((N_PAD, HID), _f32),
    pltpu.VMEM_SHARED((N_PAD,), _f32),
]

_G_DNUMS = lax.GatherDimensionNumbers(
    offset_dims=(), collapsed_slice_dims=(0,), start_index_map=(0,))


def _sc_edge_body(srcr, dstr, hc, asc, adc, hr, asr, adr, m2, z2, z1,
                  pc, dc, pr, dr,
                  srcv, dstv, mv, *rest):
    set0, set1 = rest[0:10], rest[10:20]
    accc, denc, accr, denr = rest[20:24]
    sets = (set0, set1)
    cid = lax.axis_index("c")
    sid = lax.axis_index("s")
    wid = sid * 2 + cid
    r0 = sid * ROWS_PER_TILE

    pltpu.sync_copy(z2.at[pl.ds(r0, ROWS_PER_TILE)], accc.at[pl.ds(r0, ROWS_PER_TILE)])
    pltpu.sync_copy(z2.at[pl.ds(r0, ROWS_PER_TILE)], accr.at[pl.ds(r0, ROWS_PER_TILE)])
    pltpu.sync_copy(z1.at[pl.ds(r0, ROWS_PER_TILE)], denc.at[pl.ds(r0, ROWS_PER_TILE)])
    pltpu.sync_copy(z1.at[pl.ds(r0, ROWS_PER_TILE)], denr.at[pl.ds(r0, ROWS_PER_TILE)])
    plsc.subcore_barrier()

    pltpu.sync_copy(srcr.at[pl.ds(wid * CHUNKS, CHUNKS)], srcv)
    pltpu.sync_copy(dstr.at[pl.ds(wid * CHUNKS, CHUNKS)], dstv)
    pltpu.sync_copy(m2, mv)
    for s in sets:
        pltpu.sync_copy(z1.at[pl.ds(0, MC)], s[6])
        pltpu.sync_copy(z1.at[pl.ds(0, MC)], s[7])

    def _fire_gathers(j, s):
        si = srcv.at[j]
        di = dstv.at[j]
        sem = s[8]
        pltpu.async_copy(asc.at[si], s[0], sem)
        pltpu.async_copy(adc.at[di], s[1], sem)
        pltpu.async_copy(asr.at[si], s[2], sem)
        pltpu.async_copy(adr.at[di], s[3], sem)
        pltpu.async_copy(hc.at[si], s[4], sem)
        pltpu.async_copy(hr.at[si], s[5], sem)

    def _drain_gathers(j, s):
        si = srcv.at[j]
        di = dstv.at[j]
        sem = s[8]
        pltpu.make_async_copy(asc.at[si], s[0], sem).wait()
        pltpu.make_async_copy(adc.at[di], s[1], sem).wait()
        pltpu.make_async_copy(asr.at[si], s[2], sem).wait()
        pltpu.make_async_copy(adr.at[di], s[3], sem).wait()
        pltpu.make_async_copy(hc.at[si], s[4], sem).wait()
        pltpu.make_async_copy(hr.at[si], s[5], sem).wait()

    def _fire_scatters(j, s):
        di = dstv.at[j]
        sem = s[9]
        pltpu.async_copy(s[4], accc.at[di], sem, add=True)
        pltpu.async_copy(s[5], accr.at[di], sem, add=True)
        pltpu.async_copy(s[6], denc.at[di], sem, add=True)
        pltpu.async_copy(s[7], denr.at[di], sem, add=True)

    def _drain_scatters(s):
        di = dstv.at[0]
        sem = s[9]
        pltpu.make_async_copy(s[4], accc.at[di], sem).wait()
        pltpu.make_async_copy(s[5], accr.at[di], sem).wait()
        pltpu.make_async_copy(s[6], denc.at[di], sem).wait()
        pltpu.make_async_copy(s[7], denr.at[di], sem).wait()

    def _compute(j, s, bi):
        ase, ade, hrow, wv = s[0 + bi], s[1 + bi], s[4 + bi // 2], s[6 + bi // 2]

        def _grp(kk, c2):
            s_ = pl.ds(kk * 16, 16)
            av = ase[s_]
            dv = ade[s_]
            z = av + dv
            z = jnp.maximum(z, 0.2 * z)
            t = mv[bi // 2, pl.ds(0, 16)] + dv
            z = z - jnp.maximum(t, 0.2 * t)
            w = jnp.exp(z)
            wv[s_] = w + wv[s_] * 0.0
            for i in range(16):
                r = kk * 16 + i
                wb = lax.gather(w, jnp.full((16, 1), i, jnp.int32), _G_DNUMS,
                                (1,), mode=lax.GatherScatterMode.PROMISE_IN_BOUNDS)
                hrow[r, pl.ds(0, 16)] = hrow[r, pl.ds(0, 16)] * wb
                hrow[r, pl.ds(16, 16)] = hrow[r, pl.ds(16, 16)] * wb
            return c2

        lax.fori_loop(0, MC // 16, _grp, 0)

    _fire_gathers(0, sets[0])

    def _pair(jj, carry):
        for b in (0, 1):
            j = 2 * jj + b
            cur = sets[b]
            nxt = sets[1 - b]

            @pl.when(j > 0)
            def _():
                _drain_scatters(nxt)

            @pl.when(j + 1 < CHUNKS)
            def _():
                _fire_gathers(j + 1, nxt)

            _drain_gathers(j, cur)
            _compute(j, cur, 0)
            _compute(j, cur, 2)
            _fire_scatters(j, cur)
        return carry

    lax.fori_loop(0, CHUNKS // 2, _pair, 0)
    _drain_scatters(sets[1])
    plsc.subcore_barrier()

    pltpu.sync_copy(accc.at[pl.ds(r0, ROWS_PER_TILE)],
                    pc.at[cid, pl.ds(r0, ROWS_PER_TILE)])
    pltpu.sync_copy(accr.at[pl.ds(r0, ROWS_PER_TILE)],
                    pr.at[cid, pl.ds(r0, ROWS_PER_TILE)])
    pltpu.sync_copy(denc.at[pl.ds(r0, ROWS_PER_TILE)],
                    dc.at[cid, pl.ds(r0, ROWS_PER_TILE)])
    pltpu.sync_copy(denr.at[pl.ds(r0, ROWS_PER_TILE)],
                    dr.at[cid, pl.ds(r0, ROWS_PER_TILE)])


@functools.cache
def _sc_edge():
    mesh = plsc.VectorSubcoreMesh(core_axis_name="c", subcore_axis_name="s",
                                  num_cores=2, num_subcores=N_TILES)
    return pl.kernel(_sc_edge_body, out_type=_SC_OUT_TYPE, mesh=mesh,
                     scratch_types=_SC_SCRATCH,
                     compiler_params=pltpu.CompilerParams(use_tc_tiling_on_sc=False))


def _bd(w):
    return jnp.kron(jnp.eye(4, dtype=_f32), w)


def _amat(att_src, att_dst):
    e4 = jnp.eye(4, dtype=_f32)
    return jnp.concatenate([jnp.kron(e4, att_src[:, None]),
                            jnp.kron(e4, att_dst[:, None])], axis=1)


def kernel(x, edge_index, params):
    pcl, prg = params['cls'], params['reg']

    xp = jnp.pad(x, ((0, N_PAD - N_NODES), (0, 0))).reshape(NF, 512)
    n_fill = E_PAD - edge_index.shape[1]
    fill_src = jnp.full((n_fill,), N_NODES, jnp.int32)
    fill_dst = N_NODES + jnp.arange(n_fill, dtype=jnp.int32) % (N_PAD - N_NODES)
    srcr = jnp.concatenate([edge_index[0], fill_src]).reshape(E_PAD // MC, MC)
    dstr = jnp.concatenate([edge_index[1], fill_dst]).reshape(E_PAD // MC, MC)
    z2 = jnp.zeros((N_PAD, HID), _f32)
    z1 = jnp.zeros((N_PAD,), _f32)
    e_m = jnp.kron(jnp.eye(4, dtype=_f32), jnp.ones((1, HID), _f32))
    sq = lambda t: t.reshape(N_PAD)
    fl = lambda t: t.reshape(2, NF, 128)
    fd = lambda t: t.reshape(2, NF, 4)

    w1c = jnp.kron(jnp.eye(4, dtype=_f32), pcl['W1'])
    w1r = jnp.kron(jnp.eye(4, dtype=_f32), prg['W1'])
    a1c = _amat(pcl['att_src1'], pcl['att_dst1'])
    a1r = _amat(prg['att_src1'], prg['att_dst1'])
    hc, hr, asc, adc, mc_, asr, adr, mr_ = _tc_pre(xp, w1c, w1r, a1c, a1r)
    m2 = jnp.concatenate([mc_, mr_], axis=0)

    pc, dc, pr, dr = _sc_edge()(srcr, dstr,
                                hc.reshape(N_PAD, HID), sq(asc), sq(adc),
                                hr.reshape(N_PAD, HID), sq(asr), sq(adr),
                                m2, z2, z1)

    b1 = jnp.stack([jnp.tile(pcl['b1'], 4), jnp.tile(prg['b1'], 4)])
    h2c, h2r, asc2, adc2, mc2, asr2, adr2, mr2 = _tc_mid(
        fl(pc), fd(dc), fl(pr), fd(dr), e_m, b1, _bd(pcl['W2']), _bd(prg['W2']),
        _amat(pcl['att_src2'], pcl['att_dst2']),
        _amat(prg['att_src2'], prg['att_dst2']),
        hc, hr, asc, adc, mc_, asr, adr, mr_)
    m2 = jnp.concatenate([mc2, mr2], axis=0)

    pc, dc, pr, dr = _sc_edge()(srcr, dstr,
                                h2c.reshape(N_PAD, HID), sq(asc2), sq(adc2),
                                h2r.reshape(N_PAD, HID), sq(asr2), sq(adr2),
                                m2, z2, z1)

    b2 = jnp.stack([jnp.tile(pcl['b2'], 4), jnp.tile(prg['b2'], 4)])
    lb = jnp.stack([jnp.tile(pcl['lin_b'], 4), jnp.tile(prg['lin_b'], 4)])
    yc, yr = _tc_post(fl(pc), fd(dc), fl(pr), fd(dr), e_m, b2,
                      _bd(pcl['lin_W']), _bd(prg['lin_W']), lb,
                      h2c, h2r, asc2, adc2, mc2, asr2, adr2, mr2)
    return (yc.reshape(N_PAD, HID)[:N_NODES],
            yr.reshape(N_PAD, HID)[:N_NODES])
